# baseline (speedup 1.0000x reference)
import functools

import jax
import jax.numpy as jnp
from jax.experimental import pallas as pl
from jax.experimental.pallas import tpu as pltpu

E = 320000
NODE_DIM = 128
EDGE_DIM = 16
GLOBAL_DIM = 128
HIDDEN_DIM = 256
N_GRAPHS = 256
REL = 128
K_CAT = 3 * NODE_DIM + REL

BLOCK_E = 6400


def _edge_mlp_body(scal_ref, src_ref, dest_ref, ea_ref, low_ref, hiw_ref,
                   lo_ref, hi_ref, u_ref, w1sde_ref, w1u_ref, b1_ref, w2_ref,
                   b2_ref, out_ref, wcat_ref, uproj_ref, tail_ref):
    pid = pl.program_id(0)

    @pl.when(pid == 0)
    def _build_tables():
        tail_ref[...] = jnp.zeros_like(tail_ref)
        wcat_ref[:3 * NODE_DIM] = w1sde_ref[...]
        up = jax.lax.dot_general(
            u_ref[...], w1u_ref[...],
            (((1,), (0,)), ((), ())), preferred_element_type=jnp.float32)
        uproj_ref[...] = (up + b1_ref[...]).astype(jnp.bfloat16)

    g0 = scal_ref[0, pid] * 16
    gend = scal_ref[1, pid]
    wcat_ref[3 * NODE_DIM:] = uproj_ref[pl.ds(g0, REL), :]

    bf16 = jnp.bfloat16
    dot = functools.partial(
        jax.lax.dot_general, dimension_numbers=(((1,), (0,)), ((), ())),
        preferred_element_type=jnp.float32)

    off = pid * BLOCK_E
    ri = jax.lax.broadcasted_iota(jnp.int32, (BLOCK_E, REL), 0)
    low = low_ref[pl.ds(pid * 8, 8), :][0:1] - off
    hiw = hiw_ref[pl.ds(pid * 8, 8), :][0:1] - off
    onehot = ((ri >= low) & (ri < hiw)).astype(bf16)

    spill = gend - g0 >= REL

    @pl.when(spill)
    def _fill_tail():
        ri2 = jax.lax.broadcasted_iota(jnp.int32, (BLOCK_E, N_GRAPHS), 0)
        gi = jax.lax.broadcasted_iota(jnp.int32, (BLOCK_E, N_GRAPHS), 1)
        toh = ((ri2 >= lo_ref[...] - off) & (ri2 < hi_ref[...] - off)
               & (gi >= g0 + REL)).astype(bf16)
        tail_ref[...] = dot(toh, uproj_ref[...])

    x = jnp.concatenate(
        [src_ref[...].astype(bf16), dest_ref[...].astype(bf16),
         ea_ref[...], onehot], axis=1)
    h = dot(x, wcat_ref[...]) + tail_ref[...]

    @pl.when(spill)
    def _clear_tail():
        tail_ref[...] = jnp.zeros_like(tail_ref)

    h = jnp.maximum(h.astype(bf16), jnp.asarray(0, bf16))
    res = dot(h, w2_ref[...])
    out_ref[...] = res.T + b2_ref[...]


def kernel(src, dest, edge_attr, u, batch, W1, b1, W2, b2):
    bf16 = jnp.bfloat16
    i32 = jnp.int32
    W1sde = jnp.zeros((3 * NODE_DIM, HIDDEN_DIM), bf16)
    W1sde = W1sde.at[:2 * NODE_DIM + EDGE_DIM].set(
        W1[:2 * NODE_DIM + EDGE_DIM].astype(bf16))
    W1u = W1[2 * NODE_DIM + EDGE_DIM:]
    W2b = W2.astype(bf16)
    b1_2d = b1.reshape(1, HIDDEN_DIM)
    b2_2d = b2.reshape(EDGE_DIM, 1)
    ea_c = jnp.pad(edge_attr.astype(bf16), ((0, 0), (0, NODE_DIM - EDGE_DIM)))
    batch32 = batch.astype(i32)
    seg = jnp.searchsorted(batch32, jnp.arange(N_GRAPHS + 1, dtype=i32),
                           side="left").astype(i32)
    lo = seg[:N_GRAPHS].reshape(1, N_GRAPHS)
    hi = seg[1:].reshape(1, N_GRAPHS)

    grid = E // BLOCK_E
    gb = jnp.minimum(batch32[::BLOCK_E] & ~15, N_GRAPHS - REL)
    ge = batch32[BLOCK_E - 1::BLOCK_E]
    scal = jnp.stack([gb // 16, ge])
    win = gb[:, None] + jnp.arange(REL + 1, dtype=i32)[None, :]
    segw = jnp.take(seg, win, axis=0)
    low = jnp.repeat(segw[:, :REL], 8, axis=0)
    hiw = jnp.repeat(segw[:, 1:], 8, axis=0)

    const = lambda i, s: (0, 0)
    out = pl.pallas_call(
        _edge_mlp_body,
        grid_spec=pltpu.PrefetchScalarGridSpec(
            num_scalar_prefetch=1,
            grid=(grid,),
            in_specs=[
                pl.BlockSpec((BLOCK_E, NODE_DIM), lambda i, s: (i, 0)),
                pl.BlockSpec((BLOCK_E, NODE_DIM), lambda i, s: (i, 0)),
                pl.BlockSpec((BLOCK_E, NODE_DIM), lambda i, s: (i, 0)),
                pl.BlockSpec((8 * E // BLOCK_E, REL), const),
                pl.BlockSpec((8 * E // BLOCK_E, REL), const),
                pl.BlockSpec((1, N_GRAPHS), const),
                pl.BlockSpec((1, N_GRAPHS), const),
                pl.BlockSpec((N_GRAPHS, GLOBAL_DIM), const),
                pl.BlockSpec((3 * NODE_DIM, HIDDEN_DIM), const),
                pl.BlockSpec((GLOBAL_DIM, HIDDEN_DIM), const),
                pl.BlockSpec((1, HIDDEN_DIM), const),
                pl.BlockSpec((HIDDEN_DIM, EDGE_DIM), const),
                pl.BlockSpec((EDGE_DIM, 1), const),
            ],
            out_specs=pl.BlockSpec((EDGE_DIM, BLOCK_E), lambda i, s: (0, i)),
            scratch_shapes=[pltpu.VMEM((K_CAT, HIDDEN_DIM), bf16),
                            pltpu.VMEM((N_GRAPHS, HIDDEN_DIM), bf16),
                            pltpu.VMEM((BLOCK_E, HIDDEN_DIM), jnp.float32)],
        ),
        out_shape=jax.ShapeDtypeStruct((EDGE_DIM, E), jnp.float32),
    )(scal, src, dest, ea_c, low, hiw, lo, hi, u, W1sde, W1u, b1_2d, W2b,
      b2_2d)
    return out.T


# speedup vs baseline: 1.3855x; 1.3855x over previous
import functools

import jax
import jax.numpy as jnp
from jax.experimental import pallas as pl
from jax.experimental.pallas import tpu as pltpu

E = 320000
NODE_DIM = 128
EDGE_DIM = 16
GLOBAL_DIM = 128
HIDDEN_DIM = 256
N_GRAPHS = 256
K_CAT = 3 * NODE_DIM + N_GRAPHS

BLOCK_E = 12800


def _edge_mlp_body(src_ref, dest_ref, ea_ref, lo_ref, hi_ref, u_ref,
                   w1sde_ref, w1u_ref, b1_ref, w2_ref, b2_ref,
                   out_ref, wcat_ref):
    pid = pl.program_id(0)

    @pl.when(pid == 0)
    def _build_wcat():
        wcat_ref[:3 * NODE_DIM] = w1sde_ref[...]
        up = jax.lax.dot_general(
            u_ref[...], w1u_ref[...],
            (((1,), (0,)), ((), ())), preferred_element_type=jnp.float32)
        wcat_ref[3 * NODE_DIM:] = (up + b1_ref[...]).astype(jnp.bfloat16)

    bf16 = jnp.bfloat16
    dot = functools.partial(
        jax.lax.dot_general, dimension_numbers=(((1,), (0,)), ((), ())),
        preferred_element_type=jnp.float32)

    ri = jax.lax.broadcasted_iota(jnp.int32, (BLOCK_E, N_GRAPHS), 0)
    off = pid * BLOCK_E
    onehot = ((ri >= lo_ref[...] - off) & (ri < hi_ref[...] - off)).astype(bf16)

    x = jnp.concatenate(
        [src_ref[...].astype(bf16), dest_ref[...].astype(bf16),
         ea_ref[...], onehot], axis=1)
    h = dot(x, wcat_ref[...]).astype(bf16)
    h = jnp.maximum(h, jnp.asarray(0, bf16))
    res = dot(h, w2_ref[...])
    out_ref[...] = res.T + b2_ref[...]


def kernel(src, dest, edge_attr, u, batch, W1, b1, W2, b2):
    bf16 = jnp.bfloat16
    W1sde = jnp.zeros((3 * NODE_DIM, HIDDEN_DIM), bf16)
    W1sde = W1sde.at[:2 * NODE_DIM + EDGE_DIM].set(
        W1[:2 * NODE_DIM + EDGE_DIM].astype(bf16))
    W1u = W1[2 * NODE_DIM + EDGE_DIM:]
    W2b = W2.astype(bf16)
    b1_2d = b1.reshape(1, HIDDEN_DIM)
    b2_2d = b2.reshape(EDGE_DIM, 1)
    ea_c = jnp.pad(edge_attr.astype(bf16), ((0, 0), (0, NODE_DIM - EDGE_DIM)))
    seg = jnp.searchsorted(batch.astype(jnp.int32),
                           jnp.arange(N_GRAPHS + 1, dtype=jnp.int32),
                           side="left").astype(jnp.int32)
    lo = seg[:N_GRAPHS].reshape(1, N_GRAPHS)
    hi = seg[1:].reshape(1, N_GRAPHS)

    grid = E // BLOCK_E
    const = lambda i: (0, 0)
    out = pl.pallas_call(
        _edge_mlp_body,
        grid=(grid,),
        in_specs=[
            pl.BlockSpec((BLOCK_E, NODE_DIM), lambda i: (i, 0)),
            pl.BlockSpec((BLOCK_E, NODE_DIM), lambda i: (i, 0)),
            pl.BlockSpec((BLOCK_E, NODE_DIM), lambda i: (i, 0)),
            pl.BlockSpec((1, N_GRAPHS), const),
            pl.BlockSpec((1, N_GRAPHS), const),
            pl.BlockSpec((N_GRAPHS, GLOBAL_DIM), const),
            pl.BlockSpec((3 * NODE_DIM, HIDDEN_DIM), const),
            pl.BlockSpec((GLOBAL_DIM, HIDDEN_DIM), const),
            pl.BlockSpec((1, HIDDEN_DIM), const),
            pl.BlockSpec((HIDDEN_DIM, EDGE_DIM), const),
            pl.BlockSpec((EDGE_DIM, 1), const),
        ],
        out_specs=pl.BlockSpec((EDGE_DIM, BLOCK_E), lambda i: (0, i)),
        out_shape=jax.ShapeDtypeStruct((EDGE_DIM, E), jnp.float32),
        scratch_shapes=[pltpu.VMEM((K_CAT, HIDDEN_DIM), jnp.bfloat16)],
    )(src, dest, ea_c, lo, hi, u, W1sde, W1u, b1_2d, W2b, b2_2d)
    return out.T
